# baseline (speedup 1.0000x reference)
import functools

import jax
import jax.numpy as jnp
from jax import lax
from jax.experimental import pallas as pl
from jax.experimental.pallas import tpu as pltpu
from jax.experimental.pallas import tpu_sc as plsc

_B, _N, _D = 16, 2048, 256
_ROWS = _B * _N
_NW = 32
_RPW = _ROWS // _NW
_CHUNK = 64
_NCH = _RPW // _CHUNK
_PAIRS = _NCH // 2
_LANES = 16
_GRP = _D // _LANES
_HGRP = _GRP // 2
_DU = _D // 2
_REP = 8
_VROWS = 513


@functools.partial(
    pl.kernel,
    mesh=plsc.VectorSubcoreMesh(core_axis_name="c", subcore_axis_name="s"),
    out_type=jax.ShapeDtypeStruct(((_N + 1) * _B, _D), jnp.float32),
    scratch_types=[
        pltpu.VMEM((_RPW,), jnp.int32),
        pltpu.VMEM((_RPW,), jnp.int32),
        pltpu.VMEM((_NCH, _CHUNK), jnp.int32),
        pltpu.VMEM((_CHUNK, _D), jnp.float32),
        pltpu.VMEM((_CHUNK, _DU), jnp.uint32),
        pltpu.VMEM((_CHUNK, _DU), jnp.uint32),
        pltpu.VMEM((_CHUNK, _D), jnp.float32),
        pltpu.VMEM((_CHUNK, _D), jnp.float32),
        pltpu.VMEM((_CHUNK, _DU), jnp.uint32),
        pltpu.VMEM((_CHUNK, _DU), jnp.uint32),
        pltpu.VMEM((_CHUNK, _D), jnp.float32),
        pltpu.VMEM((_LANES, _D), jnp.float32),
        pltpu.SemaphoreType.DMA,
        pltpu.SemaphoreType.DMA,
        pltpu.SemaphoreType.DMA,
        pltpu.SemaphoreType.DMA,
        pltpu.SemaphoreType.DMA,
        pltpu.SemaphoreType.DMA,
        pltpu.SemaphoreType.DMA,
        pltpu.SemaphoreType.DMA,
        pltpu.SemaphoreType.DMA,
    ],
)
def _sc_node_feature(feat_hbm, idxin_hbm, idxout_hbm, inw_hbm, outw_hbm,
                     gt_hbm, out_hbm,
                     idxin_v, idxout_v, orow_v,
                     feat_a, inr_a, outr_a, st_a,
                     feat_b, inr_b, outr_b, st_b,
                     gt_v,
                     sf_a, si_a, so_a, sst_a,
                     sf_b, si_b, so_b, sst_b, sgt):
    c = lax.axis_index("c")
    s = lax.axis_index("s")
    wid = s * 2 + c
    base = wid * _RPW
    batch = wid // 2
    half = wid % 2
    n0 = half * _RPW
    orow0 = 1 + n0

    sets = (
        (feat_a, inr_a, outr_a, st_a, sf_a, si_a, so_a, sst_a),
        (feat_b, inr_b, outr_b, st_b, sf_b, si_b, so_b, sst_b),
    )

    pltpu.sync_copy(idxin_hbm.at[pl.ds(base, _RPW)], idxin_v)
    pltpu.sync_copy(idxout_hbm.at[pl.ds(base, _RPW)], idxout_v)

    rep_off = (wid % _REP) * _VROWS

    def rep_body(i, carry):
        sl = pl.ds(i * _LANES, _LANES)
        idxin_v[sl] = idxin_v[sl] + rep_off
        idxout_v[sl] = idxout_v[sl] + rep_off
        return carry
    lax.fori_loop(0, _RPW // _LANES, rep_body, 0)

    lane = lax.iota(jnp.int32, _LANES)

    def orow_body(k, carry):
        for jj in range(_CHUNK // _LANES):
            orow_v[k, pl.ds(jj * _LANES, _LANES)] = (
                (orow0 + k * _CHUNK + jj * _LANES + lane) * _B + batch)
        return carry
    lax.fori_loop(0, _NCH, orow_body, 0)

    @pl.when(wid == 0)
    def _():
        pltpu.sync_copy(gt_hbm, gt_v.at[pl.ds(0, 1)])
        for j in range(_GRP):
            sl = pl.ds(j * _LANES, _LANES)
            row = gt_v[0, sl]
            for r in range(1, _LANES):
                gt_v[r, sl] = row
        pltpu.async_copy(gt_v, out_hbm.at[lane], sgt).wait()

    def start_in(ci, fb, ib, ob, sf, si, so):
        pltpu.async_copy(
            feat_hbm.at[batch].at[pl.ds(n0 + ci * _CHUNK, _CHUNK)], fb, sf)
        pltpu.async_copy(inw_hbm.at[idxin_v.at[pl.ds(ci * _CHUNK, _CHUNK)]],
                         ib, si)
        pltpu.async_copy(outw_hbm.at[idxout_v.at[pl.ds(ci * _CHUNK, _CHUNK)]],
                         ob, so)

    def wait_in(ci, fb, ib, ob, sf, si, so):
        pltpu.make_async_copy(
            feat_hbm.at[batch].at[pl.ds(n0 + ci * _CHUNK, _CHUNK)],
            fb, sf).wait()
        pltpu.make_async_copy(
            inw_hbm.at[idxin_v.at[pl.ds(ci * _CHUNK, _CHUNK)]],
            ib, si).wait()
        pltpu.make_async_copy(
            outw_hbm.at[idxout_v.at[pl.ds(ci * _CHUNK, _CHUNK)]],
            ob, so).wait()

    def out_ref(ci):
        return out_hbm.at[orow_v.at[ci]]

    hi_mask = jnp.full((_LANES,), 0xFFFF0000, jnp.uint32)
    sh16 = jnp.full((_LANES,), 16, jnp.uint32)

    def compute(fb, ib, ob, sb):
        def row_body(r, carry):
            for g in range(_HGRP):
                psl = pl.ds(g * _LANES, _LANES)
                u_i = ib[r, psl]
                u_o = ob[r, psl]
                a_i = lax.bitcast_convert_type(lax.shift_left(u_i, sh16), jnp.float32)
                b_i = lax.bitcast_convert_type(u_i & hi_mask, jnp.float32)
                a_o = lax.bitcast_convert_type(lax.shift_left(u_o, sh16), jnp.float32)
                b_o = lax.bitcast_convert_type(u_o & hi_mask, jnp.float32)
                sl0 = pl.ds(32 * g, _LANES)
                sl1 = pl.ds(32 * g + _LANES, _LANES)
                sb[r, sl0] = fb[r, sl0] + a_i + a_o
                sb[r, sl1] = fb[r, sl1] + b_i + b_o
            return carry
        lax.fori_loop(0, _CHUNK, row_body, 0)

    start_in(0, *sets[0][:3], *sets[0][4:7])
    start_in(1, *sets[1][:3], *sets[1][4:7])

    def pair_body(p, carry):
        for b in (0, 1):
            fb, ib, ob, sb, sf, si, so, sst = sets[b]
            ci = p * 2 + b
            wait_in(ci, fb, ib, ob, sf, si, so)

            @pl.when(p > 0)
            def _():
                pltpu.make_async_copy(sb, out_ref(ci - 2), sst).wait()

            compute(fb, ib, ob, sb)
            pltpu.async_copy(sb, out_ref(ci), sst)

            @pl.when(p < _PAIRS - 1)
            def _():
                start_in(ci + 2, fb, ib, ob, sf, si, so)
        return carry
    lax.fori_loop(0, _PAIRS, pair_body, 0)

    pltpu.make_async_copy(st_a, out_ref(_NCH - 2), sst_a).wait()
    pltpu.make_async_copy(st_b, out_ref(_NCH - 1), sst_b).wait()


def _pack_table(w):
    w16 = w.astype(jnp.bfloat16).reshape(_VROWS, _HGRP, 2, _LANES)
    w16 = w16.transpose(0, 1, 3, 2)
    u = jax.lax.bitcast_convert_type(w16, jnp.uint32).reshape(_VROWS, _DU)
    return jnp.tile(u, (_REP, 1))


def kernel(features, in_degree, out_degree, in_w, out_w, graph_token):
    idx_in = in_degree.astype(jnp.int32).reshape(_ROWS)
    idx_out = out_degree.astype(jnp.int32).reshape(_ROWS)
    out = _sc_node_feature(features, idx_in, idx_out, _pack_table(in_w),
                           _pack_table(out_w), graph_token)
    return out.reshape(_N + 1, _B, _D).transpose(1, 0, 2)


# speedup vs baseline: 1.0237x; 1.0237x over previous
import functools

import jax
import jax.numpy as jnp
from jax import lax
from jax.experimental import pallas as pl
from jax.experimental.pallas import tpu as pltpu
from jax.experimental.pallas import tpu_sc as plsc

_B, _N, _D = 16, 2048, 256
_ROWS = _B * _N
_NW = 32
_RPW = _ROWS // _NW
_CHUNK = 32
_NCH = _RPW // _CHUNK
_PAIRS = _NCH // 2
_LANES = 16
_GRP = _D // _LANES
_HGRP = _GRP // 2
_DU = _D // 2
_REP = 8
_VROWS = 513


@functools.partial(
    pl.kernel,
    mesh=plsc.VectorSubcoreMesh(core_axis_name="c", subcore_axis_name="s"),
    out_type=jax.ShapeDtypeStruct(((_N + 1) * _B, _D), jnp.float32),
    scratch_types=[
        pltpu.VMEM((_RPW,), jnp.int32),
        pltpu.VMEM((_RPW,), jnp.int32),
        pltpu.VMEM((_NCH, _CHUNK), jnp.int32),
        pltpu.VMEM((_CHUNK, _D), jnp.float32),
        pltpu.VMEM((_CHUNK, _DU), jnp.uint32),
        pltpu.VMEM((_CHUNK, _DU), jnp.uint32),
        pltpu.VMEM((_CHUNK, _D), jnp.float32),
        pltpu.VMEM((_CHUNK, _D), jnp.float32),
        pltpu.VMEM((_CHUNK, _DU), jnp.uint32),
        pltpu.VMEM((_CHUNK, _DU), jnp.uint32),
        pltpu.VMEM((_CHUNK, _D), jnp.float32),
        pltpu.VMEM((_LANES, _D), jnp.float32),
        pltpu.SemaphoreType.DMA,
        pltpu.SemaphoreType.DMA,
        pltpu.SemaphoreType.DMA,
        pltpu.SemaphoreType.DMA,
        pltpu.SemaphoreType.DMA,
        pltpu.SemaphoreType.DMA,
        pltpu.SemaphoreType.DMA,
        pltpu.SemaphoreType.DMA,
        pltpu.SemaphoreType.DMA,
    ],
)
def _sc_node_feature(feat_hbm, idxin_hbm, idxout_hbm, inw_hbm, outw_hbm,
                     gt_hbm, out_hbm,
                     idxin_v, idxout_v, orow_v,
                     feat_a, inr_a, outr_a, st_a,
                     feat_b, inr_b, outr_b, st_b,
                     gt_v,
                     sf_a, si_a, so_a, sst_a,
                     sf_b, si_b, so_b, sst_b, sgt):
    c = lax.axis_index("c")
    s = lax.axis_index("s")
    wid = s * 2 + c
    base = wid * _RPW
    batch = wid // 2
    half = wid % 2
    n0 = half * _RPW
    orow0 = 1 + n0

    sets = (
        (feat_a, inr_a, outr_a, st_a, sf_a, si_a, so_a, sst_a),
        (feat_b, inr_b, outr_b, st_b, sf_b, si_b, so_b, sst_b),
    )

    pltpu.sync_copy(idxin_hbm.at[pl.ds(base, _RPW)], idxin_v)
    pltpu.sync_copy(idxout_hbm.at[pl.ds(base, _RPW)], idxout_v)

    rep_off = (wid % _REP) * _VROWS

    def rep_body(i, carry):
        sl = pl.ds(i * _LANES, _LANES)
        idxin_v[sl] = idxin_v[sl] + rep_off
        idxout_v[sl] = idxout_v[sl] + rep_off
        return carry
    lax.fori_loop(0, _RPW // _LANES, rep_body, 0)

    lane = lax.iota(jnp.int32, _LANES)

    def orow_body(k, carry):
        for jj in range(_CHUNK // _LANES):
            orow_v[k, pl.ds(jj * _LANES, _LANES)] = (
                (orow0 + k * _CHUNK + jj * _LANES + lane) * _B + batch)
        return carry
    lax.fori_loop(0, _NCH, orow_body, 0)

    @pl.when(wid == 0)
    def _():
        pltpu.sync_copy(gt_hbm, gt_v.at[pl.ds(0, 1)])
        for j in range(_GRP):
            sl = pl.ds(j * _LANES, _LANES)
            row = gt_v[0, sl]
            for r in range(1, _LANES):
                gt_v[r, sl] = row
        pltpu.async_copy(gt_v, out_hbm.at[lane], sgt).wait()

    def start_in(ci, fb, ib, ob, sf, si, so):
        pltpu.async_copy(
            feat_hbm.at[batch].at[pl.ds(n0 + ci * _CHUNK, _CHUNK)], fb, sf)
        pltpu.async_copy(inw_hbm.at[idxin_v.at[pl.ds(ci * _CHUNK, _CHUNK)]],
                         ib, si)
        pltpu.async_copy(outw_hbm.at[idxout_v.at[pl.ds(ci * _CHUNK, _CHUNK)]],
                         ob, so)

    def wait_in(ci, fb, ib, ob, sf, si, so):
        pltpu.make_async_copy(
            feat_hbm.at[batch].at[pl.ds(n0 + ci * _CHUNK, _CHUNK)],
            fb, sf).wait()
        pltpu.make_async_copy(
            inw_hbm.at[idxin_v.at[pl.ds(ci * _CHUNK, _CHUNK)]],
            ib, si).wait()
        pltpu.make_async_copy(
            outw_hbm.at[idxout_v.at[pl.ds(ci * _CHUNK, _CHUNK)]],
            ob, so).wait()

    def out_ref(ci):
        return out_hbm.at[orow_v.at[ci]]

    hi_mask = jnp.full((_LANES,), 0xFFFF0000, jnp.uint32)
    sh16 = jnp.full((_LANES,), 16, jnp.uint32)

    def compute(fb, ib, ob, sb):
        def row_body(r, carry):
            for g in range(_HGRP):
                psl = pl.ds(g * _LANES, _LANES)
                u_i = ib[r, psl]
                u_o = ob[r, psl]
                a_i = lax.bitcast_convert_type(lax.shift_left(u_i, sh16), jnp.float32)
                b_i = lax.bitcast_convert_type(u_i & hi_mask, jnp.float32)
                a_o = lax.bitcast_convert_type(lax.shift_left(u_o, sh16), jnp.float32)
                b_o = lax.bitcast_convert_type(u_o & hi_mask, jnp.float32)
                sl0 = pl.ds(32 * g, _LANES)
                sl1 = pl.ds(32 * g + _LANES, _LANES)
                sb[r, sl0] = fb[r, sl0] + a_i + a_o
                sb[r, sl1] = fb[r, sl1] + b_i + b_o
            return carry
        lax.fori_loop(0, _CHUNK, row_body, 0)

    start_in(0, *sets[0][:3], *sets[0][4:7])
    start_in(1, *sets[1][:3], *sets[1][4:7])

    def pair_body(p, carry):
        for b in (0, 1):
            fb, ib, ob, sb, sf, si, so, sst = sets[b]
            ci = p * 2 + b
            wait_in(ci, fb, ib, ob, sf, si, so)

            @pl.when(p > 0)
            def _():
                pltpu.make_async_copy(sb, out_ref(ci - 2), sst).wait()

            compute(fb, ib, ob, sb)
            pltpu.async_copy(sb, out_ref(ci), sst)

            @pl.when(p < _PAIRS - 1)
            def _():
                start_in(ci + 2, fb, ib, ob, sf, si, so)
        return carry
    lax.fori_loop(0, _PAIRS, pair_body, 0)

    pltpu.make_async_copy(st_a, out_ref(_NCH - 2), sst_a).wait()
    pltpu.make_async_copy(st_b, out_ref(_NCH - 1), sst_b).wait()


def _pack_table(w):
    w16 = w.astype(jnp.bfloat16).reshape(_VROWS, _HGRP, 2, _LANES)
    w16 = w16.transpose(0, 1, 3, 2)
    u = jax.lax.bitcast_convert_type(w16, jnp.uint32).reshape(_VROWS, _DU)
    return jnp.tile(u, (_REP, 1))


def kernel(features, in_degree, out_degree, in_w, out_w, graph_token):
    idx_in = in_degree.astype(jnp.int32).reshape(_ROWS)
    idx_out = out_degree.astype(jnp.int32).reshape(_ROWS)
    out = _sc_node_feature(features, idx_in, idx_out, _pack_table(in_w),
                           _pack_table(out_w), graph_token)
    return out.reshape(_N + 1, _B, _D).transpose(1, 0, 2)


# speedup vs baseline: 1.1661x; 1.1391x over previous
import functools

import jax
import jax.numpy as jnp
from jax import lax
from jax.experimental import pallas as pl
from jax.experimental.pallas import tpu as pltpu
from jax.experimental.pallas import tpu_sc as plsc

_B, _N, _D = 16, 2048, 256
_ROWS = _B * _N
_NW = 32
_RPW = _ROWS // _NW
_CHUNK = 32
_NCH = _RPW // _CHUNK
_NSET = 4
_LANES = 16
_GRP = _D // _LANES
_REP = 8
_VROWS = 513

_BUF = [pltpu.VMEM((_CHUNK, _D), jnp.float32)] * (3 * _NSET)
_SEM = [pltpu.SemaphoreType.DMA] * (4 * _NSET + 1)


@functools.partial(
    pl.kernel,
    mesh=plsc.VectorSubcoreMesh(core_axis_name="c", subcore_axis_name="s"),
    out_type=jax.ShapeDtypeStruct(((_N + 1) * _B, _D), jnp.float32),
    scratch_types=[
        pltpu.VMEM((_RPW,), jnp.int32),
        pltpu.VMEM((_RPW,), jnp.int32),
        pltpu.VMEM((_NCH, _CHUNK), jnp.int32),
        pltpu.VMEM((_LANES, _D), jnp.float32),
    ] + _BUF + _SEM,
)
def _sc_node_feature(feat_hbm, idxin_hbm, idxout_hbm, inw_hbm, outw_hbm,
                     gt_hbm, out_hbm,
                     idxin_v, idxout_v, orow_v, gt_v, *bufs_and_sems):
    bufs = bufs_and_sems[:3 * _NSET]
    sems = bufs_and_sems[3 * _NSET:]
    sgt = sems[4 * _NSET]
    sets = tuple(
        (bufs[3 * i], bufs[3 * i + 1], bufs[3 * i + 2],
         sems[4 * i], sems[4 * i + 1], sems[4 * i + 2], sems[4 * i + 3])
        for i in range(_NSET))

    c = lax.axis_index("c")
    s = lax.axis_index("s")
    wid = s * 2 + c
    base = wid * _RPW
    batch = wid // 2
    half = wid % 2
    n0 = half * _RPW
    orow0 = 1 + n0

    pltpu.sync_copy(idxin_hbm.at[pl.ds(base, _RPW)], idxin_v)
    pltpu.sync_copy(idxout_hbm.at[pl.ds(base, _RPW)], idxout_v)

    rep_off = (wid % _REP) * _VROWS

    def rep_body(i, carry):
        sl = pl.ds(i * _LANES, _LANES)
        idxin_v[sl] = idxin_v[sl] + rep_off
        idxout_v[sl] = idxout_v[sl] + rep_off
        return carry
    lax.fori_loop(0, _RPW // _LANES, rep_body, 0)

    lane = lax.iota(jnp.int32, _LANES)

    def orow_body(k, carry):
        for jj in range(_CHUNK // _LANES):
            orow_v[k, pl.ds(jj * _LANES, _LANES)] = (
                (orow0 + k * _CHUNK + jj * _LANES + lane) * _B + batch)
        return carry
    lax.fori_loop(0, _NCH, orow_body, 0)

    @pl.when(wid == 0)
    def _():
        pltpu.sync_copy(gt_hbm, gt_v.at[pl.ds(0, 1)])
        for j in range(_GRP):
            sl = pl.ds(j * _LANES, _LANES)
            row = gt_v[0, sl]
            for r in range(1, _LANES):
                gt_v[r, sl] = row
        pltpu.async_copy(gt_v, out_hbm.at[lane], sgt).wait()

    def start_in(ci, st):
        fb, ib, ob = st[0], st[1], st[2]
        sf, si, so = st[3], st[4], st[5]
        pltpu.async_copy(
            feat_hbm.at[batch].at[pl.ds(n0 + ci * _CHUNK, _CHUNK)], fb, sf)
        pltpu.async_copy(inw_hbm.at[idxin_v.at[pl.ds(ci * _CHUNK, _CHUNK)]],
                         ib, si)
        pltpu.async_copy(outw_hbm.at[idxout_v.at[pl.ds(ci * _CHUNK, _CHUNK)]],
                         ob, so)

    def wait_in(ci, st):
        fb, ib, ob = st[0], st[1], st[2]
        sf, si, so = st[3], st[4], st[5]
        pltpu.make_async_copy(
            feat_hbm.at[batch].at[pl.ds(n0 + ci * _CHUNK, _CHUNK)],
            fb, sf).wait()
        pltpu.make_async_copy(
            inw_hbm.at[idxin_v.at[pl.ds(ci * _CHUNK, _CHUNK)]],
            ib, si).wait()
        pltpu.make_async_copy(
            outw_hbm.at[idxout_v.at[pl.ds(ci * _CHUNK, _CHUNK)]],
            ob, so).wait()

    def out_ref(ci):
        return out_hbm.at[orow_v.at[ci]]

    def wait_store(ci, st):
        pltpu.make_async_copy(st[0], out_ref(ci), st[6]).wait()

    def compute(st):
        fb, ib, ob = st[0], st[1], st[2]

        def row_body(r, carry):
            for j in range(_GRP):
                sl = pl.ds(j * _LANES, _LANES)
                fb[r, sl] = fb[r, sl] + ib[r, sl] + ob[r, sl]
            return carry
        lax.fori_loop(0, _CHUNK, row_body, 0)

    for b in range(_NSET):
        start_in(b, sets[b])

    def group_body(p, carry):
        for b in range(_NSET):
            ci = p * _NSET + b
            st = sets[b]
            wait_in(ci, st)
            compute(st)
            pltpu.async_copy(st[0], out_ref(ci), st[6])
            nxt = ci + _NSET - 1
            st_n = sets[(b + _NSET - 1) % _NSET]
            if b == 0:
                @pl.when(p > 0)
                def _():
                    wait_store(ci - 1, st_n)
                    start_in(nxt, st_n)
            else:
                @pl.when(p < _NCH // _NSET - 1)
                def _():
                    wait_store(ci - 1, st_n)
                    start_in(nxt, st_n)
        return carry
    lax.fori_loop(0, _NCH // _NSET, group_body, 0)

    for b in range(_NSET):
        ci = _NCH - _NSET + b
        wait_store(ci, sets[b])


def kernel(features, in_degree, out_degree, in_w, out_w, graph_token):
    idx_in = in_degree.astype(jnp.int32).reshape(_ROWS)
    idx_out = out_degree.astype(jnp.int32).reshape(_ROWS)
    in_w_rep = jnp.tile(in_w, (_REP, 1))
    out_w_rep = jnp.tile(out_w, (_REP, 1))
    out = _sc_node_feature(features, idx_in, idx_out, in_w_rep, out_w_rep,
                           graph_token)
    return out.reshape(_N + 1, _B, _D).transpose(1, 0, 2)


# speedup vs baseline: 1.1755x; 1.0080x over previous
import functools

import jax
import jax.numpy as jnp
from jax import lax
from jax.experimental import pallas as pl
from jax.experimental.pallas import tpu as pltpu
from jax.experimental.pallas import tpu_sc as plsc

_B, _N, _D = 16, 2048, 256
_ROWS = _B * _N
_NW = 32
_RPW = _ROWS // _NW
_CHUNK = 32
_NCH = _RPW // _CHUNK
_NSET = 4
_LANES = 16
_GRP = _D // _LANES
_REP = 4
_VROWS = 513

_BUF = [pltpu.VMEM((_CHUNK, _D), jnp.float32)] * (3 * _NSET)
_SEM = [pltpu.SemaphoreType.DMA] * (4 * _NSET + 1)


@functools.partial(
    pl.kernel,
    mesh=plsc.VectorSubcoreMesh(core_axis_name="c", subcore_axis_name="s"),
    out_type=jax.ShapeDtypeStruct(((_N + 1) * _B, _D), jnp.float32),
    scratch_types=[
        pltpu.VMEM((_RPW,), jnp.int32),
        pltpu.VMEM((_RPW,), jnp.int32),
        pltpu.VMEM((_NCH, _CHUNK), jnp.int32),
        pltpu.VMEM((_LANES, _D), jnp.float32),
    ] + _BUF + _SEM,
)
def _sc_node_feature(feat_hbm, idxin_hbm, idxout_hbm, inw_hbm, outw_hbm,
                     gt_hbm, out_hbm,
                     idxin_v, idxout_v, orow_v, gt_v, *bufs_and_sems):
    bufs = bufs_and_sems[:3 * _NSET]
    sems = bufs_and_sems[3 * _NSET:]
    sgt = sems[4 * _NSET]
    sets = tuple(
        (bufs[3 * i], bufs[3 * i + 1], bufs[3 * i + 2],
         sems[4 * i], sems[4 * i + 1], sems[4 * i + 2], sems[4 * i + 3])
        for i in range(_NSET))

    c = lax.axis_index("c")
    s = lax.axis_index("s")
    wid = s * 2 + c
    base = wid * _RPW
    batch = wid // 2
    half = wid % 2
    n0 = half * _RPW
    orow0 = 1 + n0

    pltpu.sync_copy(idxin_hbm.at[pl.ds(base, _RPW)], idxin_v)
    pltpu.sync_copy(idxout_hbm.at[pl.ds(base, _RPW)], idxout_v)

    rep_off = (wid % _REP) * _VROWS

    def rep_body(i, carry):
        sl = pl.ds(i * _LANES, _LANES)
        idxin_v[sl] = idxin_v[sl] + rep_off
        idxout_v[sl] = idxout_v[sl] + rep_off
        return carry
    lax.fori_loop(0, _RPW // _LANES, rep_body, 0)

    lane = lax.iota(jnp.int32, _LANES)

    def orow_body(k, carry):
        for jj in range(_CHUNK // _LANES):
            orow_v[k, pl.ds(jj * _LANES, _LANES)] = (
                (orow0 + k * _CHUNK + jj * _LANES + lane) * _B + batch)
        return carry
    lax.fori_loop(0, _NCH, orow_body, 0)

    @pl.when(wid == 0)
    def _():
        pltpu.sync_copy(gt_hbm, gt_v.at[pl.ds(0, 1)])
        for j in range(_GRP):
            sl = pl.ds(j * _LANES, _LANES)
            row = gt_v[0, sl]
            for r in range(1, _LANES):
                gt_v[r, sl] = row
        pltpu.async_copy(gt_v, out_hbm.at[lane], sgt).wait()

    def start_in(ci, st):
        fb, ib, ob = st[0], st[1], st[2]
        sf, si, so = st[3], st[4], st[5]
        pltpu.async_copy(
            feat_hbm.at[batch].at[pl.ds(n0 + ci * _CHUNK, _CHUNK)], fb, sf)
        pltpu.async_copy(inw_hbm.at[idxin_v.at[pl.ds(ci * _CHUNK, _CHUNK)]],
                         ib, si)
        pltpu.async_copy(outw_hbm.at[idxout_v.at[pl.ds(ci * _CHUNK, _CHUNK)]],
                         ob, so)

    def wait_in(ci, st):
        fb, ib, ob = st[0], st[1], st[2]
        sf, si, so = st[3], st[4], st[5]
        pltpu.make_async_copy(
            feat_hbm.at[batch].at[pl.ds(n0 + ci * _CHUNK, _CHUNK)],
            fb, sf).wait()
        pltpu.make_async_copy(
            inw_hbm.at[idxin_v.at[pl.ds(ci * _CHUNK, _CHUNK)]],
            ib, si).wait()
        pltpu.make_async_copy(
            outw_hbm.at[idxout_v.at[pl.ds(ci * _CHUNK, _CHUNK)]],
            ob, so).wait()

    def out_ref(ci):
        return out_hbm.at[orow_v.at[ci]]

    def wait_store(ci, st):
        pltpu.make_async_copy(st[0], out_ref(ci), st[6]).wait()

    def compute(st):
        fb, ib, ob = st[0], st[1], st[2]

        def row_body(r, carry):
            for j in range(_GRP):
                sl = pl.ds(j * _LANES, _LANES)
                fb[r, sl] = fb[r, sl] + ib[r, sl] + ob[r, sl]
            return carry
        lax.fori_loop(0, _CHUNK, row_body, 0)

    for b in range(_NSET):
        start_in(b, sets[b])

    def group_body(p, carry):
        for b in range(_NSET):
            ci = p * _NSET + b
            st = sets[b]
            wait_in(ci, st)
            compute(st)
            pltpu.async_copy(st[0], out_ref(ci), st[6])
            nxt = ci + _NSET - 1
            st_n = sets[(b + _NSET - 1) % _NSET]
            if b == 0:
                @pl.when(p > 0)
                def _():
                    wait_store(ci - 1, st_n)
                    start_in(nxt, st_n)
            else:
                @pl.when(p < _NCH // _NSET - 1)
                def _():
                    wait_store(ci - 1, st_n)
                    start_in(nxt, st_n)
        return carry
    lax.fori_loop(0, _NCH // _NSET, group_body, 0)

    for b in range(_NSET):
        ci = _NCH - _NSET + b
        wait_store(ci, sets[b])


def kernel(features, in_degree, out_degree, in_w, out_w, graph_token):
    idx_in = in_degree.astype(jnp.int32).reshape(_ROWS)
    idx_out = out_degree.astype(jnp.int32).reshape(_ROWS)
    in_w_rep = jnp.tile(in_w, (_REP, 1))
    out_w_rep = jnp.tile(out_w, (_REP, 1))
    out = _sc_node_feature(features, idx_in, idx_out, in_w_rep, out_w_rep,
                           graph_token)
    return out.reshape(_N + 1, _B, _D).transpose(1, 0, 2)
